# baseline (speedup 1.0000x reference)
import jax
import jax.numpy as jnp
from jax import lax
from jax.experimental import pallas as pl
from jax.experimental.pallas import tpu as pltpu
from jax.experimental.pallas import tpu_sc as plsc

NUM_FIELDS = 26
VOCAB = 100000
EMBED_DIM = 32
BATCH = 16384

NC = 2
NS = 16
NW = NC * NS
L = 16

BB = 128
NU = NUM_FIELDS * (BATCH // BB)
UPW = NU // NW
DT = EMBED_DIM // 8
RPR = 128 // EMBED_DIM
NREC = NUM_FIELDS * VOCAB // RPR


def _body(xq_hbm, tab_hbm, out_hbm, idx_v, sub_v, rows, trs, gsems, wsems):
    w = lax.axis_index("s") * NC + lax.axis_index("c")
    u0 = w * UPW

    pltpu.sync_copy(xq_hbm.at[pl.ds(u0, UPW)], idx_v)

    def split_idx(j):
        f = lax.shift_right_logical(u0 + j, 7)
        roff = f * (VOCAB // RPR)
        for q in range(BB // L):
            v = idx_v[j, pl.ds(q * L, L)]
            sub_v[j, pl.ds(q * L, L)] = lax.rem(v, RPR)
            idx_v[j, pl.ds(q * L, L)] = (
                lax.shift_right_logical(v, 2) + roff
            )

    def start_gather(j, par):
        pltpu.async_copy(tab_hbm.at[idx_v.at[j]], rows[par], gsems[par])

    def wait_gather(par):
        pltpu.make_async_copy(tab_hbm.at[idx_v.at[0]], rows[par],
                              gsems[par]).wait()

    bidx = [lax.iota(jnp.int32, L) + bb * L for bb in range(BB // L)]

    def transpose(j, par):
        def tr_d(d, _):
            r = lax.shift_right_logical(d, 3)
            i = lax.rem(d, 8)
            for bb in range(BB // L):
                col = sub_v[j, pl.ds(bb * L, L)] * EMBED_DIM + d
                v = plsc.load_gather(rows[par], [bidx[bb], col])
                trs[par][r, i, pl.ds(bb * L, L)] = v
            return 0

        lax.fori_loop(0, EMBED_DIM, tr_d, 0)

    def start_write(j, par):
        u = u0 + j
        f = lax.shift_right_logical(u, 7)
        c = lax.rem(u, BB)
        pltpu.async_copy(trs[par], out_hbm.at[f, :, c], wsems[par])

    def wait_write(par):
        pltpu.make_async_copy(trs[par], out_hbm.at[0, :, 0],
                              wsems[par]).wait()

    split_idx(0)
    start_gather(0, 0)

    def step(k, _):
        for par in range(2):
            u = k * 2 + par

            @pl.when(u + 1 < UPW)
            def _():
                split_idx(u + 1)
                start_gather(u + 1, 1 - par)

            wait_gather(par)

            @pl.when(u >= 2)
            def _():
                wait_write(par)

            transpose(u, par)
            start_write(u, par)
        return 0

    lax.fori_loop(0, UPW // 2, step, 0)
    wait_write(0)
    wait_write(1)


@jax.jit
def _embed(x_cat, tables):
    xq = jnp.transpose(x_cat.astype(jnp.int32)).reshape(NU, BB)
    tab_rec = tables.reshape(NREC, RPR * EMBED_DIM)
    mesh = plsc.VectorSubcoreMesh(core_axis_name="c", subcore_axis_name="s")
    f = pl.kernel(
        _body,
        out_type=jax.ShapeDtypeStruct(
            (NUM_FIELDS, DT, BATCH // BB, 8, BB), jnp.float32
        ),
        mesh=mesh,
        scratch_types=[
            pltpu.VMEM((UPW, BB), jnp.int32),
            pltpu.VMEM((UPW, BB), jnp.int32),
            [pltpu.VMEM((BB, RPR * EMBED_DIM), jnp.float32) for _ in range(2)],
            [pltpu.VMEM((DT, 8, BB), jnp.float32) for _ in range(2)],
            [pltpu.SemaphoreType.DMA for _ in range(2)],
            [pltpu.SemaphoreType.DMA for _ in range(2)],
        ],
        compiler_params=pltpu.CompilerParams(
            use_tc_tiling_on_sc=True, needs_layout_passes=False
        ),
    )
    out5d = f(xq, tab_rec)
    return jnp.transpose(out5d, (2, 4, 0, 1, 3)).reshape(
        BATCH, NUM_FIELDS, EMBED_DIM
    )


def kernel(x_cat, tables):
    return _embed(x_cat, tables)


# speedup vs baseline: 1.0740x; 1.0740x over previous
import jax
import jax.numpy as jnp
from jax import lax
from jax.experimental import pallas as pl
from jax.experimental.pallas import tpu as pltpu
from jax.experimental.pallas import tpu_sc as plsc

NUM_FIELDS = 26
VOCAB = 100000
EMBED_DIM = 32
BATCH = 16384

NC = 2
NS = 16
NW = NC * NS
L = 16

BB = 128
NU = NUM_FIELDS * (BATCH // BB)
UPW = NU // NW
DT = EMBED_DIM // 8
NR = 4


def _body(xq_hbm, tab_hbm, out_hbm, idx_v, rows, trs, gsems, wsems):
    w = lax.axis_index("s") * NC + lax.axis_index("c")
    u0 = w * UPW

    pltpu.sync_copy(xq_hbm.at[pl.ds(u0, UPW)], idx_v)

    def add_offset(j):
        f = lax.shift_right_logical(u0 + j, 7)
        off = f * VOCAB
        for q in range(BB // L):
            idx_v[j, pl.ds(q * L, L)] = idx_v[j, pl.ds(q * L, L)] + off

    def start_gather(j, s):
        pltpu.async_copy(
            tab_hbm.at[idx_v.at[j, pl.ds(0, BB // 2)]],
            rows[s].at[pl.ds(0, BB // 2)], gsems[s],
        )
        pltpu.async_copy(
            tab_hbm.at[idx_v.at[j, pl.ds(BB // 2, BB // 2)]],
            rows[s].at[pl.ds(BB // 2, BB // 2)], gsems[s],
        )

    def wait_gather(s):
        pltpu.make_async_copy(tab_hbm.at[idx_v.at[0]], rows[s],
                              gsems[s]).wait()

    bidx = [lax.iota(jnp.int32, L) + bb * L for bb in range(BB // L)]
    dvecs = [jnp.full((L,), d, jnp.int32) for d in range(EMBED_DIM)]

    def transpose(s):
        for d in range(EMBED_DIM):
            r, i = d >> 3, d & 7
            for bb in range(BB // L):
                v = plsc.load_gather(rows[s], [bidx[bb], dvecs[d]])
                trs[s][r, i, pl.ds(bb * L, L)] = v

    def start_write(j, s):
        u = u0 + j
        f = lax.shift_right_logical(u, 7)
        c = lax.rem(u, BB)
        pltpu.async_copy(trs[s], out_hbm.at[f, :, c], wsems[s])

    def wait_write(s):
        pltpu.make_async_copy(trs[s], out_hbm.at[0, :, 0],
                              wsems[s]).wait()

    for j in range(NR - 1):
        add_offset(j)
        start_gather(j, j)

    def step(k, _):
        for par in range(NR):
            u = k * NR + par

            @pl.when(u + NR - 1 < UPW)
            def _():
                add_offset(u + NR - 1)
                start_gather(u + NR - 1, (par + NR - 1) % NR)

            wait_gather(par)

            @pl.when(u >= NR)
            def _():
                wait_write(par)

            transpose(par)
            start_write(u, par)
        return 0

    lax.fori_loop(0, UPW // NR, step, 0)
    for s in range(NR):
        wait_write(s)


@jax.jit
def _embed(x_cat, tables):
    xq = jnp.transpose(x_cat.astype(jnp.int32)).reshape(NU, BB)
    tab_flat = tables.reshape(NUM_FIELDS * VOCAB, EMBED_DIM)
    mesh = plsc.VectorSubcoreMesh(core_axis_name="c", subcore_axis_name="s")
    f = pl.kernel(
        _body,
        out_type=jax.ShapeDtypeStruct(
            (NUM_FIELDS, DT, BATCH // BB, 8, BB), jnp.float32
        ),
        mesh=mesh,
        scratch_types=[
            pltpu.VMEM((UPW, BB), jnp.int32),
            [pltpu.VMEM((BB, EMBED_DIM), jnp.float32) for _ in range(NR)],
            [pltpu.VMEM((DT, 8, BB), jnp.float32) for _ in range(NR)],
            [pltpu.SemaphoreType.DMA for _ in range(NR)],
            [pltpu.SemaphoreType.DMA for _ in range(NR)],
        ],
        compiler_params=pltpu.CompilerParams(
            use_tc_tiling_on_sc=False, needs_layout_passes=False
        ),
    )
    out5d = f(xq, tab_flat)
    return jnp.transpose(out5d, (2, 4, 0, 1, 3)).reshape(
        BATCH, NUM_FIELDS, EMBED_DIM
    )


def kernel(x_cat, tables):
    return _embed(x_cat, tables)


# speedup vs baseline: 5.0378x; 4.6908x over previous
import jax
import jax.numpy as jnp
from jax import lax
from jax.experimental import pallas as pl
from jax.experimental.pallas import tpu as pltpu
from jax.experimental.pallas import tpu_sc as plsc

NUM_FIELDS = 26
VOCAB = 100000
EMBED_DIM = 32
BATCH = 16384

NC = 2
NS = 16
NW = NC * NS
L = 16

BB = 128
NU = NUM_FIELDS * (BATCH // BB)
QB = BATCH // 4
DT = EMBED_DIM // 8


def _body(xq_hbm, tab_hbm, out_hbm, idx_v, row_v, outq, gsem, isem, wsems):
    w = lax.axis_index("s") * NC + lax.axis_index("c")
    r = lax.shift_right_logical(w, 3)
    i = lax.rem(w, 8)

    def field(f, _):
        ic = pltpu.async_copy(
            xq_hbm.at[pl.ds(f * (BATCH // BB), BATCH // BB)], idx_v, isem
        )
        gc = pltpu.async_copy(tab_hbm.at[f, w], row_v, gsem)
        ic.wait()
        gc.wait()

        for q in range(4):
            for c in range(QB // BB):
                base = q * (QB // BB) + c
                for bb in range(BB // L):
                    v = idx_v[base, pl.ds(bb * L, L)]
                    outq[q % 2, c, pl.ds(bb * L, L)] = plsc.load_gather(
                        row_v, [v]
                    )

            @pl.when((f > 0) | (q >= 2))
            def _():
                pltpu.make_async_copy(
                    outq.at[q % 2], out_hbm.at[0, 0, pl.ds(0, QB // BB), 0],
                    wsems[q % 2],
                ).wait()

            pltpu.async_copy(
                outq.at[q % 2],
                out_hbm.at[f, r, pl.ds(q * (QB // BB), QB // BB), i],
                wsems[q % 2],
            )
        return 0

    lax.fori_loop(0, NUM_FIELDS, field, 0)
    for s in range(2):
        pltpu.make_async_copy(
            outq.at[s], out_hbm.at[0, 0, pl.ds(0, QB // BB), 0], wsems[s]
        ).wait()


@jax.jit
def _embed(x_cat, tables):
    xq = jnp.transpose(x_cat.astype(jnp.int32)).reshape(NU, BB)
    tab_t = jnp.transpose(tables, (0, 2, 1))
    mesh = plsc.VectorSubcoreMesh(core_axis_name="c", subcore_axis_name="s")
    f = pl.kernel(
        _body,
        out_type=jax.ShapeDtypeStruct(
            (NUM_FIELDS, DT, BATCH // BB, 8, BB), jnp.float32
        ),
        mesh=mesh,
        scratch_types=[
            pltpu.VMEM((BATCH // BB, BB), jnp.int32),
            pltpu.VMEM((VOCAB,), jnp.float32),
            pltpu.VMEM((2, QB // BB, BB), jnp.float32),
            pltpu.SemaphoreType.DMA,
            pltpu.SemaphoreType.DMA,
            [pltpu.SemaphoreType.DMA for _ in range(2)],
        ],
        compiler_params=pltpu.CompilerParams(
            use_tc_tiling_on_sc=True, needs_layout_passes=False
        ),
    )
    out5d = f(xq, tab_t)
    return jnp.transpose(out5d, (2, 4, 0, 1, 3)).reshape(
        BATCH, NUM_FIELDS, EMBED_DIM
    )


def kernel(x_cat, tables):
    return _embed(x_cat, tables)


# speedup vs baseline: 5.5533x; 1.1023x over previous
import jax
import jax.numpy as jnp
from jax import lax
from jax.experimental import pallas as pl
from jax.experimental.pallas import tpu as pltpu
from jax.experimental.pallas import tpu_sc as plsc

NUM_FIELDS = 26
VOCAB = 100000
EMBED_DIM = 32
BATCH = 16384

NC = 2
NS = 16
NW = NC * NS
L = 16

BB = 128
NU = NUM_FIELDS * (BATCH // BB)
QB = BATCH // 4
DT = EMBED_DIM // 8


ROWSPL = (0, 24960, 49920, 75008, VOCAB)


def _body(xq_hbm, tab_hbm, out_hbm, idx_v, row_v, outq, gsem, isem, wsems):
    w = lax.axis_index("s") * NC + lax.axis_index("c")
    r = lax.shift_right_logical(w, 3)
    i = lax.rem(w, 8)

    def start_idx(f, q, s):
        pltpu.async_copy(
            xq_hbm.at[pl.ds(f * (BATCH // BB) + q * (QB // BB), QB // BB)],
            idx_v.at[s], isem,
        )

    def wait_idx(s):
        pltpu.make_async_copy(
            xq_hbm.at[pl.ds(0, QB // BB)], idx_v.at[s], isem
        ).wait()

    start_idx(0, 0, 0)

    def field(f, _):
        pltpu.async_copy(tab_hbm.at[f, w], row_v, gsem)

        for q in range(4):
            wait_idx(q % 2)

            if q < 3:
                start_idx(f, q + 1, (q + 1) % 2)
            else:

                @pl.when(f + 1 < NUM_FIELDS)
                def _():
                    start_idx(f + 1, 0, (q + 1) % 2)

            if q == 0:
                pltpu.make_async_copy(
                    tab_hbm.at[0, 0], row_v, gsem
                ).wait()

            for c in range(QB // BB):
                for bb in range(BB // L):
                    v = idx_v[q % 2, c, pl.ds(bb * L, L)]
                    outq[q % 2, c, pl.ds(bb * L, L)] = plsc.load_gather(
                        row_v, [v]
                    )

            @pl.when((f > 0) | (q >= 2))
            def _():
                pltpu.make_async_copy(
                    outq.at[q % 2], out_hbm.at[0, 0, pl.ds(0, QB // BB), 0],
                    wsems[q % 2],
                ).wait()

            pltpu.async_copy(
                outq.at[q % 2],
                out_hbm.at[f, r, pl.ds(q * (QB // BB), QB // BB), i],
                wsems[q % 2],
            )
        return 0

    lax.fori_loop(0, NUM_FIELDS, field, 0)
    for s in range(2):
        pltpu.make_async_copy(
            outq.at[s], out_hbm.at[0, 0, pl.ds(0, QB // BB), 0], wsems[s]
        ).wait()


@jax.jit
def _embed(x_cat, tables):
    xq = jnp.transpose(x_cat.astype(jnp.int32)).reshape(NU, BB)
    tab_t = jnp.transpose(tables, (0, 2, 1))
    mesh = plsc.VectorSubcoreMesh(core_axis_name="c", subcore_axis_name="s")
    f = pl.kernel(
        _body,
        out_type=jax.ShapeDtypeStruct(
            (NUM_FIELDS, DT, BATCH // BB, 8, BB), jnp.float32
        ),
        mesh=mesh,
        scratch_types=[
            pltpu.VMEM((2, QB // BB, BB), jnp.int32),
            pltpu.VMEM((VOCAB,), jnp.float32),
            pltpu.VMEM((2, QB // BB, BB), jnp.float32),
            pltpu.SemaphoreType.DMA,
            pltpu.SemaphoreType.DMA,
            [pltpu.SemaphoreType.DMA for _ in range(2)],
        ],
        compiler_params=pltpu.CompilerParams(
            use_tc_tiling_on_sc=True, needs_layout_passes=False
        ),
    )
    out5d = f(xq, tab_t)
    return jnp.transpose(out5d, (2, 4, 0, 1, 3)).reshape(
        BATCH, NUM_FIELDS, EMBED_DIM
    )


def kernel(x_cat, tables):
    return _embed(x_cat, tables)
